# baseline (speedup 1.0000x reference)
import functools

import jax
import jax.numpy as jnp
from jax import lax
from jax.experimental import pallas as pl
from jax.experimental.pallas import tpu as pltpu
from jax.experimental.pallas import tpu_sc as plsc

N_NODE = 100000
EMBED_DIM = 128
CHUNK = 128
NUM_WORKERS = 32
FULL_CHUNKS = N_NODE // CHUNK
TAIL = N_NODE - FULL_CHUNKS * CHUNK
TAIL_BASE = FULL_CHUNKS * CHUNK
MAX_SLOTS = 25
MAX_Z_ROWS = 37
NBUF = 6

_mesh = plsc.VectorSubcoreMesh(core_axis_name="c", subcore_axis_name="s")


@functools.partial(
    pl.kernel,
    mesh=_mesh,
    out_type=jax.ShapeDtypeStruct((N_NODE, EMBED_DIM), jnp.float32),
    scratch_types=[
        pltpu.VMEM((MAX_SLOTS * CHUNK,), jnp.int32),
        pltpu.VMEM((NBUF * CHUNK, EMBED_DIM), jnp.float32),
        pltpu.VMEM((TAIL,), jnp.int32),
        pltpu.VMEM((TAIL, EMBED_DIM), jnp.float32),
        pltpu.VMEM_SHARED((MAX_Z_ROWS, EMBED_DIM), jnp.float32),
        pltpu.SemaphoreType.DMA,
        pltpu.SemaphoreType.DMA,
        pltpu.SemaphoreType.DMA,
        pltpu.SemaphoreType.DMA,
        pltpu.SemaphoreType.DMA,
        pltpu.SemaphoreType.DMA,
        pltpu.SemaphoreType.DMA,
        pltpu.SemaphoreType.DMA,
        pltpu.SemaphoreType.DMA,
        pltpu.SemaphoreType.DMA,
        pltpu.SemaphoreType.DMA,
        pltpu.SemaphoreType.DMA,
        pltpu.SemaphoreType.DMA,
        pltpu.SemaphoreType.DMA,
    ],
)
def _embed_sc(z_hbm, w_hbm, out_hbm, idx_all, rbuf, idx_t, rows_t, w_sh,
              isem, tsem, g0, g1, g2, g3, g4, g5, w0, w1, w2, w3, w4, w5):
    gsem = (g0, g1, g2, g3, g4, g5)
    wsem = (w0, w1, w2, w3, w4, w5)
    wid = lax.axis_index("s") * 2 + lax.axis_index("c")

    @pl.when(lax.axis_index("s") == 0)
    def _():
        pltpu.sync_copy(w_hbm, w_sh)

    plsc.subcore_barrier()

    def buf(b):
        return rbuf.at[pl.ds(b * CHUNK, CHUNK)]

    def g_start(s, b):
        return pltpu.async_copy(
            w_sh.at[idx_all.at[pl.ds(s * CHUNK, CHUNK)]], buf(b), gsem[b])

    def g_wait(b):
        pltpu.make_async_copy(
            out_hbm.at[pl.ds(0, CHUNK)], buf(b), gsem[b]).wait()

    def w_start(s, b):
        return pltpu.async_copy(
            buf(b), out_hbm.at[pl.ds((wid + s * NUM_WORKERS) * CHUNK, CHUNK)],
            wsem[b])

    def w_wait(b):
        pltpu.make_async_copy(
            buf(b), out_hbm.at[pl.ds(0, CHUNK)], wsem[b]).wait()

    preload = [
        pltpu.async_copy(
            z_hbm.at[pl.ds((wid + s * NUM_WORKERS) * CHUNK, CHUNK)],
            idx_all.at[pl.ds(s * CHUNK, CHUNK)], isem)
        for s in range(MAX_SLOTS - 1)
    ]
    for cp in preload:
        cp.wait()

    @pl.when(wid < FULL_CHUNKS - (MAX_SLOTS - 1) * NUM_WORKERS)
    def _():
        s = MAX_SLOTS - 1
        pltpu.sync_copy(
            z_hbm.at[pl.ds((wid + s * NUM_WORKERS) * CHUNK, CHUNK)],
            idx_all.at[pl.ds(s * CHUNK, CHUNK)])

    for s in range(NBUF - 1):
        g_start(s, s)

    for s in range(NBUF - 1):
        g_wait(s)
        w_start(s, s)
        if s >= 1:
            w_wait(s - 1)
        g_start(s + NBUF - 1, (s + NBUF - 1) % NBUF)

    def loop_body(it, carry):
        for j in range(NBUF):
            b = (NBUF - 1 + j) % NBUF
            nb = (NBUF - 2 + j) % NBUF
            s = (NBUF - 1) + it * NBUF + j
            g_wait(b)
            w_start(s, b)

            @pl.when(wid + (s + NBUF - 1) * NUM_WORKERS < FULL_CHUNKS)
            def _():
                w_wait(nb)
                g_start(s + NBUF - 1, nb)

        return carry

    lax.fori_loop(0, (23 - (NBUF - 1)) // NBUF, loop_body, 0)

    g_wait(23 % NBUF)
    w_start(23, 23 % NBUF)

    @pl.when(wid < FULL_CHUNKS - (MAX_SLOTS - 1) * NUM_WORKERS)
    def _():
        g_wait(24 % NBUF)
        w_start(24, 24 % NBUF)

    @pl.when(wid == NUM_WORKERS - 1)
    def _():
        pltpu.sync_copy(z_hbm.at[pl.ds(TAIL_BASE, TAIL)], idx_t)
        pltpu.async_copy(w_sh.at[idx_t], rows_t, tsem).wait()
        pltpu.sync_copy(rows_t, out_hbm.at[pl.ds(TAIL_BASE, TAIL)])

    for b in range(NBUF):
        w_wait(b)


def kernel(z, z_embed_weight):
    return _embed_sc(z.astype(jnp.int32), z_embed_weight)


# speedup vs baseline: 1.0195x; 1.0195x over previous
import functools

import jax
import jax.numpy as jnp
from jax import lax
from jax.experimental import pallas as pl
from jax.experimental.pallas import tpu as pltpu
from jax.experimental.pallas import tpu_sc as plsc

N_NODE = 100000
EMBED_DIM = 128
CHUNK = 128
NUM_WORKERS = 32
FULL_CHUNKS = N_NODE // CHUNK
TAIL = N_NODE - FULL_CHUNKS * CHUNK
TAIL_BASE = FULL_CHUNKS * CHUNK
MAX_SLOTS = 25
MAX_Z_ROWS = 37
NBUF = 6

_mesh = plsc.VectorSubcoreMesh(core_axis_name="c", subcore_axis_name="s")


@functools.partial(
    pl.kernel,
    mesh=_mesh,
    out_type=jax.ShapeDtypeStruct((N_NODE, EMBED_DIM), jnp.float32),
    scratch_types=[
        pltpu.VMEM((MAX_SLOTS * CHUNK,), jnp.int32),
        pltpu.VMEM((NBUF * CHUNK, EMBED_DIM), jnp.float32),
        pltpu.VMEM((TAIL,), jnp.int32),
        pltpu.VMEM((TAIL, EMBED_DIM), jnp.float32),
        pltpu.VMEM_SHARED((MAX_Z_ROWS, EMBED_DIM), jnp.float32),
        pltpu.SemaphoreType.DMA,
        pltpu.SemaphoreType.DMA,
        pltpu.SemaphoreType.DMA,
        pltpu.SemaphoreType.DMA,
        pltpu.SemaphoreType.DMA,
        pltpu.SemaphoreType.DMA,
        pltpu.SemaphoreType.DMA,
        pltpu.SemaphoreType.DMA,
        pltpu.SemaphoreType.DMA,
        pltpu.SemaphoreType.DMA,
        pltpu.SemaphoreType.DMA,
        pltpu.SemaphoreType.DMA,
        pltpu.SemaphoreType.DMA,
        pltpu.SemaphoreType.DMA,
    ],
)
def _embed_sc(z_hbm, w_hbm, out_hbm, idx_all, rbuf, idx_t, rows_t, w_sh,
              isem, tsem, g0, g1, g2, g3, g4, g5, w0, w1, w2, w3, w4, w5):
    gsem = (g0, g1, g2, g3, g4, g5)
    wsem = (w0, w1, w2, w3, w4, w5)
    wid = lax.axis_index("s") * 2 + lax.axis_index("c")

    preload = [
        pltpu.async_copy(
            z_hbm.at[pl.ds((wid + s * NUM_WORKERS) * CHUNK, CHUNK)],
            idx_all.at[pl.ds(s * CHUNK, CHUNK)], isem)
        for s in range(MAX_SLOTS - 1)
    ]

    @pl.when(lax.axis_index("s") == 0)
    def _():
        pltpu.sync_copy(w_hbm, w_sh)

    @pl.when(wid < FULL_CHUNKS - (MAX_SLOTS - 1) * NUM_WORKERS)
    def _():
        s = MAX_SLOTS - 1
        pltpu.sync_copy(
            z_hbm.at[pl.ds((wid + s * NUM_WORKERS) * CHUNK, CHUNK)],
            idx_all.at[pl.ds(s * CHUNK, CHUNK)])

    for cp in preload:
        cp.wait()

    plsc.subcore_barrier()

    def buf(b):
        return rbuf.at[pl.ds(b * CHUNK, CHUNK)]

    def g_start(s, b):
        return pltpu.async_copy(
            w_sh.at[idx_all.at[pl.ds(s * CHUNK, CHUNK)]], buf(b), gsem[b])

    def g_wait(b):
        pltpu.make_async_copy(
            out_hbm.at[pl.ds(0, CHUNK)], buf(b), gsem[b]).wait()

    def w_start(s, b):
        return pltpu.async_copy(
            buf(b), out_hbm.at[pl.ds((wid + s * NUM_WORKERS) * CHUNK, CHUNK)],
            wsem[b])

    def w_wait(b):
        pltpu.make_async_copy(
            buf(b), out_hbm.at[pl.ds(0, CHUNK)], wsem[b]).wait()

    for s in range(NBUF - 1):
        g_start(s, s)

    for s in range(NBUF - 1):
        g_wait(s)
        w_start(s, s)
        if s >= 1:
            w_wait(s - 1)
        g_start(s + NBUF - 1, (s + NBUF - 1) % NBUF)

    def loop_body(it, carry):
        for j in range(NBUF):
            b = (NBUF - 1 + j) % NBUF
            nb = (NBUF - 2 + j) % NBUF
            s = (NBUF - 1) + it * NBUF + j
            g_wait(b)
            w_start(s, b)

            @pl.when(wid + (s + NBUF - 1) * NUM_WORKERS < FULL_CHUNKS)
            def _():
                w_wait(nb)
                g_start(s + NBUF - 1, nb)

        return carry

    lax.fori_loop(0, (23 - (NBUF - 1)) // NBUF, loop_body, 0)

    g_wait(23 % NBUF)
    w_start(23, 23 % NBUF)

    @pl.when(wid < FULL_CHUNKS - (MAX_SLOTS - 1) * NUM_WORKERS)
    def _():
        g_wait(24 % NBUF)
        w_start(24, 24 % NBUF)

    @pl.when(wid == NUM_WORKERS - 1)
    def _():
        pltpu.sync_copy(z_hbm.at[pl.ds(TAIL_BASE, TAIL)], idx_t)
        pltpu.async_copy(w_sh.at[idx_t], rows_t, tsem).wait()
        pltpu.sync_copy(rows_t, out_hbm.at[pl.ds(TAIL_BASE, TAIL)])

    for b in range(NBUF):
        w_wait(b)


def kernel(z, z_embed_weight):
    return _embed_sc(z.astype(jnp.int32), z_embed_weight)
